# baseline (speedup 1.0000x reference)
import jax
import jax.numpy as jnp
from jax.experimental import pallas as pl
from jax.experimental.pallas import tpu as pltpu

D_MODEL = 1024
FF_DIM = 512
N_ROUTED = 8
N_EXPERTS = 9
INV_DENOM = 1.0 / 3.0


def _dot_t(a, b):
    return jax.lax.dot_general(
        a, b, (((1,), (1,)), ((), ())),
        preferred_element_type=jnp.float32,
        precision=jax.lax.Precision.HIGHEST,
    )


def _moe_body(x_ref, wts_ref, w1_ref, w3_ref, w2_ref, o_ref):
    e = pl.program_id(1)
    xb = x_ref[...]

    h = jax.nn.silu(_dot_t(xb, w1_ref[0])) * _dot_t(xb, w3_ref[0])
    y = _dot_t(h, w2_ref[0])

    wts = wts_ref[...]
    lane = jax.lax.broadcasted_iota(jnp.int32, wts.shape, 1)
    col = jnp.sum(jnp.where(lane == e - 1, wts, 0.0), axis=-1, keepdims=True)
    scale = jnp.where(e == 0, 1.0, col) * INV_DENOM

    @pl.when(e == 0)
    def _init():
        o_ref[...] = y * scale

    @pl.when(e > 0)
    def _acc():
        o_ref[...] = o_ref[...] + y * scale


def kernel(x, Wg, Ws1, Ws2, Ws3, Wr1, Wr2, Wr3):
    bs, seq_len, d = x.shape
    x_flat = x.reshape(-1, d)
    T = x_flat.shape[0]

    gates = jax.nn.sigmoid(x_flat @ Wg.T)
    top_k_vals, top_k_indices = jax.lax.top_k(gates, 2)
    top_k_vals = top_k_vals / jnp.sum(top_k_vals, axis=-1, keepdims=True)
    weights = jnp.zeros((T, N_ROUTED), dtype=x_flat.dtype)
    for k in range(2):
        mask = jax.nn.one_hot(top_k_indices[:, k], N_ROUTED, dtype=x_flat.dtype)
        weights = weights + top_k_vals[:, k:k + 1] * mask

    W1 = jnp.concatenate([Ws1, Wr1], axis=0)
    W3 = jnp.concatenate([Ws3, Wr3], axis=0)
    W2 = jnp.concatenate([Ws2, Wr2], axis=0)

    TT = 1024
    out = pl.pallas_call(
        _moe_body,
        grid=(T // TT, N_EXPERTS),
        in_specs=[
            pl.BlockSpec((TT, d), lambda t, e: (t, 0)),
            pl.BlockSpec((TT, N_ROUTED), lambda t, e: (t, 0)),
            pl.BlockSpec((1, FF_DIM, d), lambda t, e: (e, 0, 0)),
            pl.BlockSpec((1, FF_DIM, d), lambda t, e: (e, 0, 0)),
            pl.BlockSpec((1, d, FF_DIM), lambda t, e: (e, 0, 0)),
        ],
        out_specs=pl.BlockSpec((TT, d), lambda t, e: (t, 0)),
        out_shape=jax.ShapeDtypeStruct((T, d), jnp.float32),
    )(x_flat, weights, W1, W3, W2)

    return out.reshape(bs, seq_len, d)


# speedup vs baseline: 3.3711x; 3.3711x over previous
import jax
import jax.numpy as jnp
from jax.experimental import pallas as pl
from jax.experimental.pallas import tpu as pltpu

D_MODEL = 1024
FF_DIM = 512
N_ROUTED = 8
N_EXPERTS = 9
INV_DENOM = 1.0 / 3.0


def _dot_t(a, b):
    return jax.lax.dot_general(
        a.astype(jnp.bfloat16), b.astype(jnp.bfloat16),
        (((1,), (1,)), ((), ())),
        preferred_element_type=jnp.float32,
    )


def _moe_body(x_ref, wts_ref, w1_ref, w3_ref, w2_ref, o_ref):
    e = pl.program_id(1)
    xb = x_ref[...]

    h = jax.nn.silu(_dot_t(xb, w1_ref[0])) * _dot_t(xb, w3_ref[0])
    y = _dot_t(h, w2_ref[0])

    wts = wts_ref[...]
    lane = jax.lax.broadcasted_iota(jnp.int32, wts.shape, 1)
    col = jnp.sum(jnp.where(lane == e - 1, wts, 0.0), axis=-1, keepdims=True)
    scale = jnp.where(e == 0, 1.0, col) * INV_DENOM

    @pl.when(e == 0)
    def _init():
        o_ref[...] = y * scale

    @pl.when(e > 0)
    def _acc():
        o_ref[...] = o_ref[...] + y * scale


def kernel(x, Wg, Ws1, Ws2, Ws3, Wr1, Wr2, Wr3):
    bs, seq_len, d = x.shape
    x_flat = x.reshape(-1, d)
    T = x_flat.shape[0]

    gates = jax.nn.sigmoid(x_flat @ Wg.T)
    top_k_vals, top_k_indices = jax.lax.top_k(gates, 2)
    top_k_vals = top_k_vals / jnp.sum(top_k_vals, axis=-1, keepdims=True)
    weights = jnp.zeros((T, N_ROUTED), dtype=x_flat.dtype)
    for k in range(2):
        mask = jax.nn.one_hot(top_k_indices[:, k], N_ROUTED, dtype=x_flat.dtype)
        weights = weights + top_k_vals[:, k:k + 1] * mask

    W1 = jnp.concatenate([Ws1, Wr1], axis=0)
    W3 = jnp.concatenate([Ws3, Wr3], axis=0)
    W2 = jnp.concatenate([Ws2, Wr2], axis=0)

    TT = 1024
    out = pl.pallas_call(
        _moe_body,
        grid=(T // TT, N_EXPERTS),
        in_specs=[
            pl.BlockSpec((TT, d), lambda t, e: (t, 0)),
            pl.BlockSpec((TT, N_ROUTED), lambda t, e: (t, 0)),
            pl.BlockSpec((1, FF_DIM, d), lambda t, e: (e, 0, 0)),
            pl.BlockSpec((1, FF_DIM, d), lambda t, e: (e, 0, 0)),
            pl.BlockSpec((1, d, FF_DIM), lambda t, e: (e, 0, 0)),
        ],
        out_specs=pl.BlockSpec((TT, d), lambda t, e: (t, 0)),
        out_shape=jax.ShapeDtypeStruct((T, d), jnp.float32),
    )(x_flat, weights, W1, W3, W2)

    return out.reshape(bs, seq_len, d)


# speedup vs baseline: 4.7002x; 1.3943x over previous
import jax
import jax.numpy as jnp
from jax.experimental import pallas as pl
from jax.experimental.pallas import tpu as pltpu

D_MODEL = 1024
FF_DIM = 512
N_ROUTED = 8
N_EXPERTS = 9
INV_DENOM = 1.0 / 3.0


def _dot_t(a, b):
    return jax.lax.dot_general(
        a.astype(jnp.bfloat16), b.astype(jnp.bfloat16),
        (((1,), (1,)), ((), ())),
        preferred_element_type=jnp.float32,
    )


def _swiglu(xb, w1, w3, w2):
    h = jax.nn.silu(_dot_t(xb, w1)) * _dot_t(xb, w3)
    return _dot_t(h, w2)


def _moe_body(x_ref, s1_ref, s3_ref, s2_ref, r1_ref, r3_ref, r2_ref,
              sc_ref, o_ref):
    e = pl.program_id(1)
    xb = x_ref[...]

    @pl.when(e == 0)
    def _shared():
        y = _swiglu(xb, s1_ref[0], s3_ref[0], s2_ref[0])
        o_ref[...] = y * INV_DENOM

    @pl.when(e > 0)
    def _routed():
        y = _swiglu(xb, r1_ref[0], r3_ref[0], r2_ref[0])
        o_ref[...] = o_ref[...] + y * sc_ref[0, 0]


def kernel(x, Wg, Ws1, Ws2, Ws3, Wr1, Wr2, Wr3):
    bs, seq_len, d = x.shape
    x_flat = x.reshape(-1, d)
    T = x_flat.shape[0]

    gates = jax.nn.sigmoid(x_flat @ Wg.T)
    top_k_vals, top_k_indices = jax.lax.top_k(gates, 2)
    top_k_vals = top_k_vals / jnp.sum(top_k_vals, axis=-1, keepdims=True)
    weights = jnp.zeros((T, N_ROUTED), dtype=x_flat.dtype)
    for k in range(2):
        mask = jax.nn.one_hot(top_k_indices[:, k], N_ROUTED, dtype=x_flat.dtype)
        weights = weights + top_k_vals[:, k:k + 1] * mask

    TT = 1024
    NT = T // TT
    scales = (weights.T * INV_DENOM).reshape(N_ROUTED, NT, TT, 1)

    re_idx = lambda t, e: (jnp.maximum(e - 1, 0), 0, 0)
    out = pl.pallas_call(
        _moe_body,
        grid=(NT, N_EXPERTS),
        in_specs=[
            pl.BlockSpec((TT, d), lambda t, e: (t, 0)),
            pl.BlockSpec((1, FF_DIM, d), lambda t, e: (0, 0, 0)),
            pl.BlockSpec((1, FF_DIM, d), lambda t, e: (0, 0, 0)),
            pl.BlockSpec((1, d, FF_DIM), lambda t, e: (0, 0, 0)),
            pl.BlockSpec((1, FF_DIM, d), re_idx),
            pl.BlockSpec((1, FF_DIM, d), re_idx),
            pl.BlockSpec((1, d, FF_DIM), re_idx),
            pl.BlockSpec((1, 1, TT, 1),
                         lambda t, e: (jnp.maximum(e - 1, 0), t, 0, 0)),
        ],
        out_specs=pl.BlockSpec((TT, d), lambda t, e: (t, 0)),
        out_shape=jax.ShapeDtypeStruct((T, d), jnp.float32),
    )(x_flat, Ws1, Ws3, Ws2, Wr1, Wr3, Wr2, scales)

    return out.reshape(bs, seq_len, d)
